# Initial kernel scaffold; baseline (speedup 1.0000x reference)
import jax
import jax.numpy as jnp
from jax.experimental import pallas as pl


def kernel(ip_feats, port_idx, tech_idx, W_ip, b_ip, port_table, tech_table):
    raise NotImplementedError("write your pallas kernel here")


# speedup vs baseline: 1.2640x; 1.2640x over previous
import functools

import jax
import jax.numpy as jnp
from jax import lax
from jax.experimental import pallas as pl
from jax.experimental.pallas import tpu as pltpu
from jax.experimental.pallas import tpu_sc as plsc

_N = 100000
_D = 64
_OUT_ROWS = 3 * _N

_CHUNK = 128
_CHUNKS = 25
_TILE_ROWS = _CHUNK * _CHUNKS
_TILE_STRIDE = 3128
_LAST_BASE = _N - _TILE_ROWS


def _sc_gather_body(port_tab, port_idx, tech_tab, tech_idx, out,
                    pidx_v, tidx_v, row_a, row_b, sem_a, sem_b):
    info = plsc.get_sparse_core_info()
    nc = info.num_cores
    wid = lax.axis_index("s") * nc + lax.axis_index("c")
    base = jnp.minimum(wid * _TILE_STRIDE, _LAST_BASE)

    pltpu.sync_copy(port_idx.at[pl.ds(base, _TILE_ROWS)], pidx_v)
    pltpu.sync_copy(tech_idx.at[pl.ds(base, _TILE_ROWS)], tidx_v)

    def step(j, _):
        off = j * _CHUNK
        pltpu.async_copy(port_tab.at[pidx_v.at[pl.ds(off, _CHUNK)]],
                         row_a, sem_a).wait()
        pltpu.sync_copy(row_a, out.at[pl.ds(_N + base + off, _CHUNK)])
        pltpu.async_copy(tech_tab.at[tidx_v.at[pl.ds(off, _CHUNK)]],
                         row_b, sem_b).wait()
        pltpu.sync_copy(row_b, out.at[pl.ds(2 * _N + base + off, _CHUNK)])
        return _

    lax.fori_loop(0, _CHUNKS, step, 0)


def _sc_gather(port_table, port_idx, tech_table, tech_idx):
    mesh = plsc.VectorSubcoreMesh(core_axis_name="c", subcore_axis_name="s")
    fn = functools.partial(
        pl.kernel,
        mesh=mesh,
        compiler_params=pltpu.CompilerParams(use_tc_tiling_on_sc=False),
        out_type=jax.ShapeDtypeStruct((_OUT_ROWS, _D), jnp.float32),
        scratch_types=[
            pltpu.VMEM((_TILE_ROWS,), jnp.int32),
            pltpu.VMEM((_TILE_ROWS,), jnp.int32),
            pltpu.VMEM((_CHUNK, _D), jnp.float32),
            pltpu.VMEM((_CHUNK, _D), jnp.float32),
            pltpu.SemaphoreType.DMA,
            pltpu.SemaphoreType.DMA,
        ],
    )(_sc_gather_body)
    return fn(port_table, port_idx, tech_table, tech_idx)


_BM = 2000


def _tc_matmul_body(alias_ref, ip_ref, w_ref, b_ref, o_ref):
    o_ref[...] = jnp.dot(ip_ref[...], w_ref[...],
                         preferred_element_type=jnp.float32) + b_ref[...]


def _tc_matmul(out_buf, ip_feats, w, b2):
    return pl.pallas_call(
        _tc_matmul_body,
        grid=(_N // _BM,),
        in_specs=[
            pl.BlockSpec((8, _D), lambda i: (0, 0)),
            pl.BlockSpec((_BM, 32), lambda i: (i, 0)),
            pl.BlockSpec((32, _D), lambda i: (0, 0)),
            pl.BlockSpec((1, _D), lambda i: (0, 0)),
        ],
        out_specs=pl.BlockSpec((_BM, _D), lambda i: (i, 0)),
        out_shape=jax.ShapeDtypeStruct((_OUT_ROWS, _D), jnp.float32),
        input_output_aliases={0: 0},
        compiler_params=pltpu.CompilerParams(
            dimension_semantics=("arbitrary",),
        ),
    )(out_buf, ip_feats, w, b2)


def kernel(ip_feats, port_idx, tech_idx, W_ip, b_ip, port_table, tech_table):
    gathered = _sc_gather(port_table, port_idx.astype(jnp.int32),
                          tech_table, tech_idx.astype(jnp.int32))
    b2 = b_ip.reshape(1, _D)
    return _tc_matmul(gathered, ip_feats, W_ip, b2)


# speedup vs baseline: 1.3990x; 1.1068x over previous
import functools

import jax
import jax.numpy as jnp
from jax import lax
from jax.experimental import pallas as pl
from jax.experimental.pallas import tpu as pltpu
from jax.experimental.pallas import tpu_sc as plsc

_N = 100000
_D = 64
_OUT_ROWS = 3 * _N

_CHUNK = 128
_CHUNKS = 25
_TILE_ROWS = _CHUNK * _CHUNKS
_TILE_STRIDE = 3128
_LAST_BASE = _N - _TILE_ROWS


def _sc_gather_body(port_tab, port_idx, tech_tab, tech_idx, out,
                    pidx_v, tidx_v, row_a, row_b, sem_a, sem_b):
    info = plsc.get_sparse_core_info()
    nc = info.num_cores
    wid = lax.axis_index("s") * nc + lax.axis_index("c")
    base = jnp.minimum(wid * _TILE_STRIDE, _LAST_BASE)

    pltpu.sync_copy(port_idx.at[pl.ds(base, _TILE_ROWS)], pidx_v)
    pltpu.sync_copy(tech_idx.at[pl.ds(base, _TILE_ROWS)], tidx_v)

    def step(j, _):
        off = j * _CHUNK
        pltpu.async_copy(port_tab.at[pidx_v.at[pl.ds(off, _CHUNK)]],
                         row_a, sem_a).wait()
        pltpu.sync_copy(row_a, out.at[pl.ds(_N + base + off, _CHUNK)])
        pltpu.async_copy(tech_tab.at[tidx_v.at[pl.ds(off, _CHUNK)]],
                         row_b, sem_b).wait()
        pltpu.sync_copy(row_b, out.at[pl.ds(2 * _N + base + off, _CHUNK)])
        return _

    lax.fori_loop(0, _CHUNKS, step, 0)


def _sc_gather(port_table, port_idx, tech_table, tech_idx):
    mesh = plsc.VectorSubcoreMesh(core_axis_name="c", subcore_axis_name="s")
    fn = functools.partial(
        pl.kernel,
        mesh=mesh,
        compiler_params=pltpu.CompilerParams(use_tc_tiling_on_sc=False),
        out_type=jax.ShapeDtypeStruct((_OUT_ROWS, _D), jnp.float32),
        scratch_types=[
            pltpu.VMEM((_TILE_ROWS,), jnp.int32),
            pltpu.VMEM((_TILE_ROWS,), jnp.int32),
            pltpu.VMEM((_CHUNK, _D), jnp.float32),
            pltpu.VMEM((_CHUNK, _D), jnp.float32),
            pltpu.SemaphoreType.DMA,
            pltpu.SemaphoreType.DMA,
        ],
    )(_sc_gather_body)
    return fn(port_table, port_idx, tech_table, tech_idx)


_BM = 1000


def _tc_matmul_body(alias_ref, ip_ref, w_ref, b_ref, o_ref):
    o_ref[...] = jnp.dot(ip_ref[...], w_ref[...],
                         preferred_element_type=jnp.float32) + b_ref[...]


def _tc_matmul(out_buf2, ip2, w2, b2):
    return pl.pallas_call(
        _tc_matmul_body,
        grid=(_N // 2 // _BM,),
        in_specs=[
            pl.BlockSpec((8, 128), lambda i: (0, 0)),
            pl.BlockSpec((_BM, _D), lambda i: (i, 0)),
            pl.BlockSpec((_D, 128), lambda i: (0, 0)),
            pl.BlockSpec((1, 128), lambda i: (0, 0)),
        ],
        out_specs=pl.BlockSpec((_BM, 128), lambda i: (i, 0)),
        out_shape=jax.ShapeDtypeStruct((_OUT_ROWS // 2, 128), jnp.float32),
        input_output_aliases={0: 0},
        compiler_params=pltpu.CompilerParams(
            dimension_semantics=("arbitrary",),
        ),
    )(out_buf2, ip2, w2, b2)


def kernel(ip_feats, port_idx, tech_idx, W_ip, b_ip, port_table, tech_table):
    gathered = _sc_gather(port_table, port_idx.astype(jnp.int32),
                          tech_table, tech_idx.astype(jnp.int32))
    g2 = gathered.reshape(_OUT_ROWS // 2, 128)
    ip2 = ip_feats.reshape(_N // 2, 64)
    w2 = jnp.zeros((_D, 128), jnp.float32)
    w2 = w2.at[0:32, 0:_D].set(W_ip).at[32:_D, _D:128].set(W_ip)
    b2 = jnp.concatenate([b_ip, b_ip]).reshape(1, 128)
    out2 = _tc_matmul(g2, ip2, w2, b2)
    return out2.reshape(_OUT_ROWS, _D)
